# Initial kernel scaffold; baseline (speedup 1.0000x reference)
import jax
import jax.numpy as jnp
from jax.experimental import pallas as pl


def kernel(feats, edge_index):
    raise NotImplementedError("write your pallas kernel here")


# speedup vs baseline: 4.2998x; 4.2998x over previous
import functools

import jax
import jax.numpy as jnp
from jax import lax
from jax.experimental import pallas as pl
from jax.experimental.pallas import tpu as pltpu, tpu_sc as plsc

NC, NS, L = 2, 16, 16
NW = NC * NS


def _k1_body(feats_h, src_h, dst_h, zeros_h, w_h, spart_h,
             src_v, dst_v, w_v, rows_s, rows_d, s_sc):
    cid = lax.axis_index("c")
    sid = lax.axis_index("s")
    wid = sid * NC + cid
    n_sub, b = src_v.shape
    n_grp = b // L
    d_feat = rows_s.shape[1]

    @pl.when(sid == 0)
    def _():
        pltpu.sync_copy(zeros_h, s_sc)
    plsc.subcore_barrier()

    pltpu.sync_copy(src_h.at[wid], src_v)
    pltpu.sync_copy(dst_h.at[wid], dst_v)

    def sub_body(c, _):
        pltpu.sync_copy(feats_h.at[src_v.at[c]], rows_s)
        pltpu.sync_copy(feats_h.at[dst_v.at[c]], rows_d)

        def grp_body(k, _):
            r16 = lax.iota(jnp.int32, L) + k * L

            def d_body(dc, acc):
                for dd in range(16):
                    d = dc * 16 + dd
                    col = jnp.full((L,), d, jnp.int32)
                    a = plsc.load_gather(rows_s, [r16, col])
                    bv = plsc.load_gather(rows_d, [r16, col])
                    acc = acc + a * bv
                return acc

            dotp = lax.fori_loop(0, d_feat // 16, d_body,
                                 jnp.zeros((L,), jnp.float32))
            w16 = jnp.exp(jnp.exp(jnp.abs(dotp) * (-0.01)))
            w_v[c, pl.ds(k * L, L)] = w16
            return 0

        lax.fori_loop(0, n_grp, grp_body, 0)
        pltpu.sync_copy(w_v.at[c], s_sc.at[dst_v.at[c]], add=True)
        return 0

    lax.fori_loop(0, n_sub, sub_body, 0)

    pltpu.sync_copy(w_v, w_h.at[wid])
    plsc.subcore_barrier()

    @pl.when(sid == 0)
    def _():
        pltpu.sync_copy(s_sc, spart_h.at[cid])


def _k2_body(spart_h, dst_h, w_h, out_h, s_a, s_b, dst_v, w_v, out_v):
    cid = lax.axis_index("c")
    sid = lax.axis_index("s")
    wid = sid * NC + cid
    n_sub, b = dst_v.shape
    n_grp = b // L
    n_nodes = s_a.shape[0]

    pltpu.sync_copy(spart_h.at[0], s_a)
    pltpu.sync_copy(spart_h.at[1], s_b)
    pltpu.sync_copy(dst_h.at[wid], dst_v)
    pltpu.sync_copy(w_h.at[wid], w_v)

    def sum_body(i, _):
        sl = pl.ds(i * L, L)
        s_a[sl] = s_a[sl] + s_b[sl]
        return 0

    lax.fori_loop(0, n_nodes // L, sum_body, 0)

    def sub_body(c, _):
        def grp_body(k, _):
            sl = pl.ds(k * L, L)
            d16 = dst_v[c, sl]
            s16 = plsc.load_gather(s_a, [d16])
            out_v[c, sl] = w_v[c, sl] / s16
            return 0

        lax.fori_loop(0, n_grp, grp_body, 0)
        return 0

    lax.fori_loop(0, n_sub, sub_body, 0)
    pltpu.sync_copy(out_v, out_h.at[wid])


def kernel(feats, edge_index):
    n_nodes, d_feat = feats.shape
    e = edge_index.shape[1]
    chunk = e // NW
    b = 80
    n_sub = chunk // b

    src3 = edge_index[0].reshape(NW, n_sub, b)
    dst3 = edge_index[1].reshape(NW, n_sub, b)
    zeros = jnp.zeros((n_nodes,), jnp.float32)

    mesh = plsc.VectorSubcoreMesh(core_axis_name="c", subcore_axis_name="s",
                                  num_cores=NC, num_subcores=NS)

    cparams = pltpu.CompilerParams(needs_layout_passes=False)

    k1 = pl.kernel(
        _k1_body,
        out_type=[
            jax.ShapeDtypeStruct((NW, n_sub, b), jnp.float32),
            jax.ShapeDtypeStruct((NC, n_nodes), jnp.float32),
        ],
        mesh=mesh,
        compiler_params=cparams,
        scratch_types=[
            pltpu.VMEM((n_sub, b), jnp.int32),
            pltpu.VMEM((n_sub, b), jnp.int32),
            pltpu.VMEM((n_sub, b), jnp.float32),
            pltpu.VMEM((b, d_feat), jnp.float32),
            pltpu.VMEM((b, d_feat), jnp.float32),
            pltpu.VMEM_SHARED((n_nodes,), jnp.float32),
        ],
    )
    w3, spart = k1(feats, src3, dst3, zeros)

    k2 = pl.kernel(
        _k2_body,
        out_type=jax.ShapeDtypeStruct((NW, n_sub, b), jnp.float32),
        mesh=mesh,
        compiler_params=cparams,
        scratch_types=[
            pltpu.VMEM((n_nodes,), jnp.float32),
            pltpu.VMEM((n_nodes,), jnp.float32),
            pltpu.VMEM((n_sub, b), jnp.int32),
            pltpu.VMEM((n_sub, b), jnp.float32),
            pltpu.VMEM((n_sub, b), jnp.float32),
        ],
    )
    out3 = k2(spart, dst3, w3)
    return out3.reshape(e, 1)
